# baseline (speedup 1.0000x reference)
import jax
import jax.numpy as jnp
from jax import lax
from jax.experimental import pallas as pl
from jax.experimental.pallas import tpu as pltpu
from jax.experimental.pallas import tpu_sc as plsc

S = 64.0
M = 0.4
H = 0.333
T_ALPHA = 0.01
EPS = 0.001

B = 1024
C = 100000
NC, NS, L = 2, 16, 16
NW = NC * NS
BPW = B // NW

_CB = 2048
_NCB = (C + _CB - 1) // _CB


def _clipn(v):
    return jnp.minimum(jnp.maximum(v, 0.001), 100.0)


def _sqrt16(x):
    tiny = 1e-20
    xc = jnp.maximum(x, tiny)
    i = plsc.bitcast(xc, jnp.int32)
    y = plsc.bitcast(0x5F3759DF - (i >> 1), jnp.float32)
    for _ in range(3):
        y = y * (1.5 - 0.5 * xc * y * y)
    r = xc * y
    r = 0.5 * (r + xc / r)
    return jnp.where(x <= tiny, 0.0, r)


def _sc_body(logits16_h, labels_h, norms_h, bm_h, bs_h,
             marg_h, nm_h, ns_h,
             lab_v, norms_v, idx_v, rows_v, marg_v, bm_v, bs_v,
             nm_v, ns_v, red_v, sem):
    wid = lax.axis_index("s") * NC + lax.axis_index("c")
    base = wid * BPW
    iota = lax.iota(jnp.int32, L)

    def _lanesum(vec):
        for k in (1, 2, 4, 8):
            red_v[...] = vec
            vec = vec + plsc.load_gather(red_v, [iota ^ k])
        return vec

    pltpu.sync_copy(labels_h.at[pl.ds(base, BPW)], lab_v)
    pltpu.sync_copy(norms_h, norms_v)
    pltpu.sync_copy(bm_h, bm_v)
    pltpu.sync_copy(bs_h, bs_v)

    def _sum_body(i, acc):
        return acc + _clipn(norms_v[pl.ds(i * L, L)])
    acc = lax.fori_loop(0, B // L, _sum_body, jnp.zeros((L,), jnp.float32))
    mean = _lanesum(acc) / B

    def _var_body(i, acc):
        d = _clipn(norms_v[pl.ds(i * L, L)]) - mean
        return acc + d * d
    acc2 = lax.fori_loop(0, B // L, _var_body, jnp.zeros((L,), jnp.float32))
    var = _lanesum(acc2) / (B - 1)
    std = _sqrt16(var)

    nm_vec = T_ALPHA * mean + (1.0 - T_ALPHA) * bm_v[...]
    ns_vec = T_ALPHA * std + (1.0 - T_ALPHA) * bs_v[...]

    for k in range(BPW // L):
        lab = lab_v[pl.ds(k * L, L)]
        rid = base + k * L + iota
        idx_v[pl.ds(k * L, L)] = (rid * C + lab) >> 4
    pltpu.async_copy(logits16_h.at[idx_v], rows_v, sem).wait()

    for k in range(BPW // L):
        lab = lab_v[pl.ds(k * L, L)]
        rid = base + k * L + iota
        off = (rid * C + lab) & 15
        t = plsc.load_gather(rows_v, [iota + k * L, off])
        n = _clipn(norms_v[pl.ds(base + k * L, L)])
        ms = jnp.clip((n - nm_vec) / (ns_vec + EPS) * H, -1.0, 1.0)
        a = M * ms
        a2 = a * a
        cos_a = 1.0 + a2 * (-0.5 + a2 * (1.0 / 24.0 + a2 * (
            -1.0 / 720.0 + a2 * (1.0 / 40320.0))))
        sin_a = a * (1.0 + a2 * (-1.0 / 6.0 + a2 * (
            1.0 / 120.0 + a2 * (-1.0 / 5040.0))))
        root = _sqrt16(1.0 - t * t)
        marg_v[pl.ds(k * L, L)] = (t * cos_a + root * sin_a - (M + a)) * S

    pltpu.sync_copy(marg_v, marg_h.at[pl.ds(base, BPW)])

    nm_v[...] = nm_vec
    ns_v[...] = ns_vec

    @pl.when(wid == 0)
    def _():
        pltpu.sync_copy(nm_v, nm_h)
        pltpu.sync_copy(ns_v, ns_h)


_sc_prep = pl.kernel(
    _sc_body,
    out_type=[
        jax.ShapeDtypeStruct((B,), jnp.float32),
        jax.ShapeDtypeStruct((L,), jnp.float32),
        jax.ShapeDtypeStruct((L,), jnp.float32),
    ],
    mesh=plsc.VectorSubcoreMesh(core_axis_name="c", subcore_axis_name="s"),
    compiler_params=pltpu.CompilerParams(
        needs_layout_passes=False, use_tc_tiling_on_sc=False),
    scratch_types=[
        pltpu.VMEM((BPW,), jnp.int32),
        pltpu.VMEM((B,), jnp.float32),
        pltpu.VMEM((BPW,), jnp.int32),
        pltpu.VMEM((BPW, L), jnp.float32),
        pltpu.VMEM((BPW,), jnp.float32),
        pltpu.VMEM((L,), jnp.float32),
        pltpu.VMEM((L,), jnp.float32),
        pltpu.VMEM((L,), jnp.float32),
        pltpu.VMEM((L,), jnp.float32),
        pltpu.VMEM((L,), jnp.float32),
        pltpu.SemaphoreType.DMA,
    ],
)


def _tc_body(lab_ref, marg_ref, x_ref, o_ref):
    j = pl.program_id(0)
    cols = lax.broadcasted_iota(jnp.int32, (B, _CB), 1) + j * _CB
    mask = cols == lab_ref[...]
    o_ref[...] = jnp.where(mask, marg_ref[...], x_ref[...] * S)


def _tc_scale(logits, labels2d, margins2d):
    return pl.pallas_call(
        _tc_body,
        grid=(_NCB,),
        in_specs=[
            pl.BlockSpec((B, 1), lambda j: (0, 0)),
            pl.BlockSpec((B, 1), lambda j: (0, 0)),
            pl.BlockSpec((B, _CB), lambda j: (0, j)),
        ],
        out_specs=pl.BlockSpec((B, _CB), lambda j: (0, j)),
        out_shape=jax.ShapeDtypeStruct((B, C), jnp.float32),
        compiler_params=pltpu.CompilerParams(
            dimension_semantics=("arbitrary",)),
    )(labels2d, margins2d, logits)


def kernel(logits, labels, norms, batch_mean, batch_std):
    logits16 = logits.reshape(B * C // L, L)
    bm16 = jnp.broadcast_to(batch_mean, (L,))
    bs16 = jnp.broadcast_to(batch_std, (L,))
    margins, nm16, ns16 = _sc_prep(
        logits16, labels, norms.reshape(B), bm16, bs16)
    out = _tc_scale(logits, labels.reshape(B, 1), margins.reshape(B, 1))
    return out, nm16[:1], ns16[:1]
